# baseline (speedup 1.0000x reference)
import functools

import jax
import jax.numpy as jnp
from jax import lax
from jax.experimental import pallas as pl
from jax.experimental.pallas import tpu as pltpu
from jax.experimental.pallas import tpu_sc as plsc

_NUM_CODES = 256
_EMBED_DIM = 768
_BATCH = 16384

_NC = 2
_NS = 16
_HALF = _EMBED_DIM // 2
_B_PER_W = _BATCH // _NS
_ROWS = 32
_NBLK = _B_PER_W // _ROWS


@functools.partial(
    pl.kernel,
    mesh=plsc.VectorSubcoreMesh(core_axis_name="c", subcore_axis_name="s"),
    out_type=jax.ShapeDtypeStruct((_BATCH, _EMBED_DIM), jnp.float32),
    scratch_types=[
        pltpu.VMEM((_NUM_CODES, _HALF), jnp.float32),
        pltpu.VMEM((_B_PER_W,), jnp.int32),
        pltpu.VMEM((2, _ROWS, _HALF), jnp.float32),
        pltpu.SemaphoreType.DMA,
        pltpu.SemaphoreType.DMA,
    ],
    compiler_params=pltpu.CompilerParams(use_tc_tiling_on_sc=False,
                                         needs_layout_passes=False),
)
def _sc_lookup(table_hbm, idx_hbm, out_hbm, tab_v, idx_v, buf_v,
               sem0, sem1):
    g = lax.axis_index("s")
    h = lax.axis_index("c")
    col0 = h * _HALF
    row_base = g * _B_PER_W
    pltpu.sync_copy(table_hbm.at[:, pl.ds(col0, _HALF)], tab_v)
    pltpu.sync_copy(idx_hbm.at[pl.ds(row_base, _B_PER_W)], idx_v)

    iota = lax.iota(jnp.int32, 16)
    sems = (sem0, sem1)
    writes = [None, None]
    for i in range(_NBLK):
        b = i % 2
        if writes[b] is not None:
            writes[b].wait()
        idx0 = idx_v[pl.ds(i * _ROWS, 16)]
        idx1 = idx_v[pl.ds(i * _ROWS + 16, 16)]

        def body(j, col, b=b, idx0=idx0, idx1=idx1):
            v0 = plsc.load_gather(tab_v, [idx0, col])
            v1 = plsc.load_gather(tab_v, [idx1, col])
            plsc.store_scatter(buf_v.at[b], [iota, col], v0)
            plsc.store_scatter(buf_v.at[b], [iota + 16, col], v1)
            return col + 1

        plsc.parallel_loop(0, _HALF, unroll=8,
                           carry=jnp.zeros((16,), jnp.int32))(body)
        writes[b] = pltpu.async_copy(
            buf_v.at[b],
            out_hbm.at[pl.ds(row_base + i * _ROWS, _ROWS),
                       pl.ds(col0, _HALF)],
            sems[b])
    writes[0].wait()
    writes[1].wait()


def kernel(anchor_embeddings, indices):
    return _sc_lookup(anchor_embeddings, indices.astype(jnp.int32))


# speedup vs baseline: 2.7105x; 2.7105x over previous
import functools

import jax
import jax.numpy as jnp
from jax import lax
from jax.experimental import pallas as pl
from jax.experimental.pallas import tpu as pltpu
from jax.experimental.pallas import tpu_sc as plsc

_NUM_CODES = 256
_EMBED_DIM = 768
_BATCH = 16384

_NC = 2
_NS = 16
_HALF = _EMBED_DIM // 2
_B_PER_W = _BATCH // _NS
_ROWS = 32
_NBLK = _B_PER_W // _ROWS


@functools.partial(
    pl.kernel,
    mesh=plsc.VectorSubcoreMesh(core_axis_name="c", subcore_axis_name="s"),
    out_type=jax.ShapeDtypeStruct((_BATCH, _EMBED_DIM), jnp.float32),
    scratch_types=[
        pltpu.VMEM((_NUM_CODES, _HALF), jnp.float32),
        pltpu.VMEM((_B_PER_W + 16,), jnp.int32),
        pltpu.VMEM((2, _ROWS, _HALF), jnp.float32),
        pltpu.SemaphoreType.DMA,
        pltpu.SemaphoreType.DMA,
    ],
    compiler_params=pltpu.CompilerParams(use_tc_tiling_on_sc=False,
                                         needs_layout_passes=False),
)
def _sc_lookup(table_hbm, idx_hbm, out_hbm, tab_v, idx_v, buf_v,
               sem0, sem1):
    g = lax.axis_index("s")
    h = lax.axis_index("c")
    col0 = h * _HALF
    row_base = g * _B_PER_W
    pltpu.sync_copy(table_hbm.at[:, pl.ds(col0, _HALF)], tab_v)
    pltpu.sync_copy(idx_hbm.at[pl.ds(row_base, _B_PER_W)],
                    idx_v.at[pl.ds(0, _B_PER_W)])

    sems = (sem0, sem1)
    _IOTA16 = lax.iota(jnp.int32, 16)
    _ZEROS16 = jnp.zeros((16,), jnp.int32)

    @pl.loop(0, _NBLK, step=2)
    def _blocks(it):
        for b in range(2):
            blk = it + b
            r0 = blk * _ROWS

            @pl.when(blk >= 2)
            def _wait(b=b):
                pltpu.make_async_copy(
                    buf_v.at[b],
                    out_hbm.at[pl.ds(row_base, _ROWS), pl.ds(col0, _HALF)],
                    sems[b]).wait()

            @plsc.parallel_loop(0, _ROWS)
            def _row(r, b=b, r0=r0):
                idxv = idx_v[pl.ds(r0 + r, 16)]
                bcast = idxv.at[_ZEROS16].get(mode="promise_in_bounds")
                dst = buf_v.at[b, r]
                for k in range(_HALF // 16):
                    v = plsc.load_gather(tab_v, [bcast, _IOTA16 + k * 16])
                    dst[pl.ds(k * 16, 16)] = v

            pltpu.async_copy(
                buf_v.at[b],
                out_hbm.at[pl.ds(row_base + r0, _ROWS), pl.ds(col0, _HALF)],
                sems[b])

    for b in range(2):
        pltpu.make_async_copy(
            buf_v.at[b],
            out_hbm.at[pl.ds(row_base, _ROWS), pl.ds(col0, _HALF)],
            sems[b]).wait()


def kernel(anchor_embeddings, indices):
    return _sc_lookup(anchor_embeddings, indices.astype(jnp.int32))


# speedup vs baseline: 5.6996x; 2.1028x over previous
import functools

import jax
import jax.numpy as jnp
from jax import lax
from jax.experimental import pallas as pl
from jax.experimental.pallas import tpu as pltpu
from jax.experimental.pallas import tpu_sc as plsc

_NUM_CODES = 256
_EMBED_DIM = 768
_BATCH = 16384

_NC = 2
_NS = 16
_HALF = _EMBED_DIM // 2
_B_PER_W = _BATCH // _NS
_ROWS = 32
_NBLK = _B_PER_W // _ROWS
_TA = _BATCH // 8
_TB = _EMBED_DIM // 128


@functools.partial(
    pl.kernel,
    mesh=plsc.VectorSubcoreMesh(core_axis_name="c", subcore_axis_name="s"),
    out_type=jax.ShapeDtypeStruct((_TA, _TB, 8, 128), jnp.float32),
    scratch_types=[
        pltpu.VMEM((_NUM_CODES, _HALF), jnp.float32),
        pltpu.VMEM((_B_PER_W + 16,), jnp.int32),
        pltpu.VMEM((2, _ROWS // 8, _HALF // 128, 8, 128), jnp.float32),
        pltpu.SemaphoreType.DMA,
        pltpu.SemaphoreType.DMA,
    ],
    compiler_params=pltpu.CompilerParams(use_tc_tiling_on_sc=False,
                                         needs_layout_passes=False),
)
def _sc_lookup(table_hbm, idx_hbm, out_hbm, tab_v, idx_v, buf_v,
               sem0, sem1):
    g = lax.axis_index("s")
    h = lax.axis_index("c")
    col0 = h * _HALF
    tb0 = h * (_HALF // 128)
    row_base = g * _B_PER_W
    pltpu.sync_copy(table_hbm.at[:, pl.ds(col0, _HALF)], tab_v)
    pltpu.sync_copy(idx_hbm.at[pl.ds(row_base, _B_PER_W)],
                    idx_v.at[pl.ds(0, _B_PER_W)])

    sems = (sem0, sem1)
    iota16 = lax.iota(jnp.int32, 16)
    zeros16 = jnp.zeros((16,), jnp.int32)

    @pl.loop(0, _NBLK, step=2)
    def _blocks(it):
        for b in range(2):
            blk = it + b
            r0 = blk * _ROWS
            ta0 = g * (_B_PER_W // 8) + blk * (_ROWS // 8)

            @pl.when(blk >= 2)
            def _wait(b=b, ta0=ta0):
                pltpu.make_async_copy(
                    buf_v.at[b],
                    out_hbm.at[pl.ds(ta0, _ROWS // 8), pl.ds(tb0, _TB // 2)],
                    sems[b]).wait()

            @plsc.parallel_loop(0, _ROWS)
            def _row(r, b=b, r0=r0):
                idxv = idx_v[pl.ds(r0 + r, 16)]
                bcast = idxv.at[zeros16].get(mode="promise_in_bounds")
                ra = r // 8
                rs = r % 8
                for k in range(_HALF // 16):
                    v = plsc.load_gather(tab_v, [bcast, iota16 + k * 16])
                    dst = buf_v.at[b, ra, k // 8, rs]
                    dst[pl.ds((k % 8) * 16, 16)] = v

            pltpu.async_copy(
                buf_v.at[b],
                out_hbm.at[pl.ds(ta0, _ROWS // 8), pl.ds(tb0, _TB // 2)],
                sems[b])

    for b in range(2):
        pltpu.make_async_copy(
            buf_v.at[b],
            out_hbm.at[pl.ds(0, _ROWS // 8), pl.ds(tb0, _TB // 2)],
            sems[b]).wait()


def kernel(anchor_embeddings, indices):
    tiled = _sc_lookup(anchor_embeddings, indices.astype(jnp.int32))
    return tiled.transpose(0, 2, 1, 3).reshape(_BATCH, _EMBED_DIM)
